# baseline (speedup 1.0000x reference)
import functools

import jax
import jax.numpy as jnp
from jax import lax
from jax.experimental import pallas as pl
from jax.experimental.pallas import tpu as pltpu
from jax.experimental.pallas import tpu_sc as plsc

_B = 16384
_NC = 2
_NS = 16
_NW = _NC * _NS
_CHUNK = 128
_BPW = _B // _NW
_NCH = _BPW // _CHUNK


def _sc_gather(route_table, node_table, route_id, node_id):
    mesh = plsc.VectorSubcoreMesh(core_axis_name="c", subcore_axis_name="s")

    @functools.partial(
        pl.kernel,
        out_type=jax.ShapeDtypeStruct((_B, 128), jnp.float32),
        mesh=mesh,
        scratch_types=[
            pltpu.VMEM((2 * _BPW,), jnp.int32),
            pltpu.VMEM((_BPW, 8), jnp.float32),
            pltpu.VMEM((_BPW, 16), jnp.float32),
            pltpu.MemorySpace.VMEM_SHARED(route_table.shape, jnp.float32),
            pltpu.MemorySpace.VMEM_SHARED(node_table.shape, jnp.float32),
            pltpu.SemaphoreType.DMA,
        ],
        compiler_params=pltpu.CompilerParams(use_tc_tiling_on_sc=False),
    )
    def gather_kernel(rtab_hbm, ntab_hbm, rid_hbm, nid_hbm, out_hbm,
                      idx_v, rrows_v, nrows_v, rtab_s, ntab_s, sem):
        sid = lax.axis_index("s")
        wid = sid * _NC + lax.axis_index("c")
        base = wid * _BPW
        ridx_copy = pltpu.async_copy(
            rid_hbm.at[pl.ds(base, _BPW)], idx_v.at[pl.ds(0, _BPW)], sem)
        nidx_copy = pltpu.async_copy(
            nid_hbm.at[pl.ds(base, _BPW)], idx_v.at[pl.ds(_BPW, _BPW)], sem)

        @pl.when(sid == 0)
        def _stage():
            pltpu.sync_copy(rtab_hbm, rtab_s)
            pltpu.sync_copy(ntab_hbm, ntab_s)

        ridx_copy.wait()
        nidx_copy.wait()
        plsc.subcore_barrier()
        copies = []
        for j in range(_NCH):
            copies.append(
                pltpu.async_copy(
                    rtab_s.at[idx_v.at[pl.ds(j * _CHUNK, _CHUNK)]],
                    rrows_v.at[pl.ds(j * _CHUNK, _CHUNK)],
                    sem,
                )
            )
            copies.append(
                pltpu.async_copy(
                    ntab_s.at[idx_v.at[pl.ds(_BPW + j * _CHUNK, _CHUNK)]],
                    nrows_v.at[pl.ds(j * _CHUNK, _CHUNK)],
                    sem,
                )
            )
        for c in copies:
            c.wait()
        pltpu.sync_copy(rrows_v,
                        out_hbm.at[pl.ds(base, _BPW), pl.ds(8, 8)])
        pltpu.sync_copy(nrows_v,
                        out_hbm.at[pl.ds(base, _BPW), pl.ds(16, 16)])

    return gather_kernel(route_table, node_table, route_id, node_id)


_TB = 4096


def _mlp_a_body(d_ref, wk_ref, wtab_ref, w1_ref, b1_ref, o_ref):
    w1wk = jnp.dot(wtab_ref[...], w1_ref[31:33],
                   preferred_element_type=jnp.float32)
    onehot = (wk_ref[...][:, None] ==
              lax.broadcasted_iota(jnp.int32, (_TB, 3), 1)).astype(jnp.float32)
    o_ref[...] = (
        jnp.dot(d_ref[...], w1_ref[0:7], preferred_element_type=jnp.float32)
        + jnp.dot(onehot, w1wk, preferred_element_type=jnp.float32)
        + b1_ref[...]
    )


def _mlp_a(dense_feats, weekday, weekday_table, W1, b1):
    grid = _B // _TB
    full = lambda shape: pl.BlockSpec(shape, lambda i: (0,) * len(shape))
    return pl.pallas_call(
        _mlp_a_body,
        grid=(grid,),
        in_specs=[
            pl.BlockSpec((_TB, 7), lambda i: (i, 0)),
            pl.BlockSpec((_TB,), lambda i: (i,)),
            full((3, 2)),
            full((33, 128)),
            full((128,)),
        ],
        out_specs=pl.BlockSpec((_TB, 128), lambda i: (i, 0)),
        out_shape=jax.ShapeDtypeStruct((_B, 128), jnp.float32),
    )(dense_feats, weekday, weekday_table, W1, b1)


def _mlp_b_body(xp_ref, p_ref, w1_ref, w2_ref, b2_ref, w3_ref, b3_ref, o_ref):
    x = xp_ref[...] + jnp.dot(p_ref[...][:, 8:32], w1_ref[7:31],
                              preferred_element_type=jnp.float32)
    h1 = jnp.maximum(x, 0.0)
    h2 = jnp.maximum(
        jnp.dot(h1, w2_ref[...], preferred_element_type=jnp.float32)
        + b2_ref[...], 0.0,
    )
    out = jnp.dot(h2, w3_ref[...], preferred_element_type=jnp.float32)
    tout = jnp.transpose(out + b3_ref[0])
    o_ref[...] = jnp.reshape(tout, (_TB // 128, 128))


def _mlp_b(xpart, pack, W1, W2, b2, W3, b3):
    grid = _B // _TB
    full = lambda shape: pl.BlockSpec(shape, lambda i: (0,) * len(shape))
    return pl.pallas_call(
        _mlp_b_body,
        grid=(grid,),
        in_specs=[
            pl.BlockSpec((_TB, 128), lambda i: (i, 0)),
            pl.BlockSpec((_TB, 128), lambda i: (i, 0)),
            full((33, 128)),
            full((128, 64)),
            full((64,)),
            full((64, 1)),
            full((1,)),
        ],
        out_specs=pl.BlockSpec((_TB // 128, 128), lambda i: (i, 0)),
        out_shape=jax.ShapeDtypeStruct((_B // 128, 128), jnp.float32),
    )(xpart, pack, W1, W2, b2, W3, b3)


def kernel(route_id, node_id, weekday, dense_feats, route_table, node_table,
           weekday_table, W1, b1, W2, b2, W3, b3):
    route_id = route_id.astype(jnp.int32)
    node_id = node_id.astype(jnp.int32)
    weekday = weekday.astype(jnp.int32)

    pack = _sc_gather(route_table, node_table, route_id, node_id)
    xpart = _mlp_a(dense_feats, weekday, weekday_table, W1, b1)
    return _mlp_b(xpart, pack, W1, W2, b2, W3, b3).reshape(_B)


# speedup vs baseline: 1.1268x; 1.1268x over previous
import functools

import jax
import jax.numpy as jnp
from jax import lax
from jax.experimental import pallas as pl
from jax.experimental.pallas import tpu as pltpu
from jax.experimental.pallas import tpu_sc as plsc

_B = 16384
_NC = 2
_NS = 16
_NW = _NC * _NS
_CHUNK = 128
_BPW = _B // _NW
_NCH = _BPW // _CHUNK


def _sc_gather(route_table, node_table, route_id, node_id):
    mesh = plsc.VectorSubcoreMesh(core_axis_name="c", subcore_axis_name="s")

    @functools.partial(
        pl.kernel,
        out_type=jax.ShapeDtypeStruct((_B, 128), jnp.float32),
        mesh=mesh,
        scratch_types=[
            pltpu.VMEM((2 * _BPW,), jnp.int32),
            pltpu.VMEM((_BPW, 8), jnp.float32),
            pltpu.VMEM((_BPW, 16), jnp.float32),
            pltpu.MemorySpace.VMEM_SHARED(route_table.shape, jnp.float32),
            pltpu.MemorySpace.VMEM_SHARED(node_table.shape, jnp.float32),
            pltpu.SemaphoreType.DMA,
        ],
        compiler_params=pltpu.CompilerParams(use_tc_tiling_on_sc=False),
    )
    def gather_kernel(rtab_hbm, ntab_hbm, rid_hbm, nid_hbm, out_hbm,
                      idx_v, rrows_v, nrows_v, rtab_s, ntab_s, sem):
        sid = lax.axis_index("s")
        wid = sid * _NC + lax.axis_index("c")
        base = wid * _BPW
        ridx_copy = pltpu.async_copy(
            rid_hbm.at[pl.ds(base, _BPW)], idx_v.at[pl.ds(0, _BPW)], sem)
        nidx_copy = pltpu.async_copy(
            nid_hbm.at[pl.ds(base, _BPW)], idx_v.at[pl.ds(_BPW, _BPW)], sem)

        @pl.when(sid == 0)
        def _stage():
            pltpu.sync_copy(rtab_hbm, rtab_s)
            pltpu.sync_copy(ntab_hbm, ntab_s)

        ridx_copy.wait()
        nidx_copy.wait()
        plsc.subcore_barrier()
        rcopy = pltpu.async_copy(
            rtab_s.at[idx_v.at[pl.ds(0, _BPW)]], rrows_v, sem)
        ncopy = pltpu.async_copy(
            ntab_s.at[idx_v.at[pl.ds(_BPW, _BPW)]], nrows_v, sem)
        rcopy.wait()
        ncopy.wait()
        pltpu.sync_copy(rrows_v,
                        out_hbm.at[pl.ds(base, _BPW), pl.ds(8, 8)])
        pltpu.sync_copy(nrows_v,
                        out_hbm.at[pl.ds(base, _BPW), pl.ds(16, 16)])

    return gather_kernel(route_table, node_table, route_id, node_id)


_TB = 4096


def _mlp_body(d_ref, p_ref, wk_ref, wtab_ref, w1_ref, b1_ref, w2_ref,
              b2_ref, w3_ref, b3_ref, o_ref):
    w1wk = jnp.dot(wtab_ref[...], w1_ref[31:33],
                   preferred_element_type=jnp.float32)
    onehot = (wk_ref[...][:, None] ==
              lax.broadcasted_iota(jnp.int32, (_TB, 3), 1)).astype(jnp.float32)
    x = (
        jnp.dot(d_ref[...], w1_ref[0:7], preferred_element_type=jnp.float32)
        + jnp.dot(p_ref[...][:, 8:32], w1_ref[7:31],
                  preferred_element_type=jnp.float32)
        + jnp.dot(onehot, w1wk, preferred_element_type=jnp.float32)
        + b1_ref[...]
    )
    h1 = jnp.maximum(x, 0.0)
    h2 = jnp.maximum(
        jnp.dot(h1, w2_ref[...], preferred_element_type=jnp.float32)
        + b2_ref[...], 0.0,
    )
    out = jnp.dot(h2, w3_ref[...], preferred_element_type=jnp.float32)
    tout = jnp.transpose(out + b3_ref[0])
    o_ref[...] = jnp.reshape(tout, (_TB // 128, 128))


def _mlp(dense_feats, pack, weekday, weekday_table, W1, b1, W2, b2, W3, b3):
    grid = _B // _TB
    full = lambda shape: pl.BlockSpec(shape, lambda i: (0,) * len(shape))
    return pl.pallas_call(
        _mlp_body,
        grid=(grid,),
        in_specs=[
            pl.BlockSpec((_TB, 7), lambda i: (i, 0)),
            pl.BlockSpec((_TB, 128), lambda i: (i, 0)),
            pl.BlockSpec((_TB,), lambda i: (i,)),
            full((3, 2)),
            full((33, 128)),
            full((128,)),
            full((128, 64)),
            full((64,)),
            full((64, 1)),
            full((1,)),
        ],
        out_specs=pl.BlockSpec((_TB // 128, 128), lambda i: (i, 0)),
        out_shape=jax.ShapeDtypeStruct((_B // 128, 128), jnp.float32),
    )(dense_feats, pack, weekday, weekday_table, W1, b1, W2, b2, W3, b3)


def kernel(route_id, node_id, weekday, dense_feats, route_table, node_table,
           weekday_table, W1, b1, W2, b2, W3, b3):
    route_id = route_id.astype(jnp.int32)
    node_id = node_id.astype(jnp.int32)
    weekday = weekday.astype(jnp.int32)

    pack = _sc_gather(route_table, node_table, route_id, node_id)
    return _mlp(dense_feats, pack, weekday, weekday_table, W1, b1, W2, b2,
                W3, b3).reshape(_B)
